# baseline (speedup 1.0000x reference)
import functools

import jax
import jax.numpy as jnp
from jax import lax
from jax.experimental import pallas as pl
from jax.experimental.pallas import tpu as pltpu
from jax.experimental.pallas import tpu_sc as plsc

N_NODES = 1_000_000
CHUNK = 8_000
N_CHUNKS = N_NODES // CHUNK
N_WORKERS = 32
MAX_ITERS = -(-N_CHUNKS // N_WORKERS)
LANES = 16
GROUPS = 4


def _sc_reduce_body(fringe_hbm, term_hbm, part_hbm,
                    f0, f1, t0, t1, acc_v, isem0, isem1):
    wid = lax.axis_index("s") * 2 + lax.axis_index("c")
    f_v, t_v = (f0, f1), (t0, t1)
    isems = (isem0, isem1)
    acc_v[...] = jnp.zeros((LANES,), jnp.float32)

    def in_copies(slot, chunk):
        off = chunk * CHUNK
        sl = pl.ds(off, CHUNK)
        return (
            pltpu.make_async_copy(fringe_hbm.at[sl], f_v[slot], isems[slot]),
            pltpu.make_async_copy(term_hbm.at[sl], t_v[slot], isems[slot]),
        )

    for it in range(MAX_ITERS):
        chunk = wid + it * N_WORKERS
        slot = it % 2

        if it == 0:
            @pl.when(chunk < N_CHUNKS)
            def _prime():
                for c in in_copies(0, chunk):
                    c.start()

        if it + 1 < MAX_ITERS:
            nxt = chunk + N_WORKERS

            @pl.when(nxt < N_CHUNKS)
            def _prefetch():
                for c in in_copies(1 - slot, nxt):
                    c.start()

        @pl.when(chunk < N_CHUNKS)
        def _process():
            for c in in_copies(slot, chunk):
                c.wait()

            zero = jnp.zeros((LANES,), jnp.float32)

            @plsc.parallel_loop(0, CHUNK, step=GROUPS * LANES, unroll=2,
                                carry=(zero, zero, zero, zero))
            def body(j, accs):
                new = []
                for g in range(GROUPS):
                    sl = pl.ds(j + g * LANES, LANES)
                    new.append(accs[g]
                               + f_v[slot][sl] * t_v[slot][sl].astype(jnp.float32))
                return tuple(new)

            a0, a1, a2, a3 = body
            acc_v[...] = acc_v[...] + ((a0 + a1) + (a2 + a3))

    pltpu.sync_copy(acc_v, part_hbm.at[wid])


@functools.partial(
    pl.kernel,
    out_type=jax.ShapeDtypeStruct((N_WORKERS, LANES), jnp.float32),
    mesh=plsc.VectorSubcoreMesh(core_axis_name="c", subcore_axis_name="s"),
    scratch_types=[
        pltpu.VMEM((CHUNK,), jnp.float32),
        pltpu.VMEM((CHUNK,), jnp.float32),
        pltpu.VMEM((CHUNK,), jnp.int32),
        pltpu.VMEM((CHUNK,), jnp.int32),
        pltpu.VMEM((LANES,), jnp.float32),
        pltpu.SemaphoreType.DMA,
        pltpu.SemaphoreType.DMA,
    ],
)
def _sc_reduce(*args):
    _sc_reduce_body(*args)


TC_BLOCK = 16_384
TC_GRID = -(-N_NODES // TC_BLOCK)


def _select_body(f_ref, m_ref, s_ref, o_ref):
    o_ref[...] = jnp.where(s_ref[...] == 0, f_ref[...], m_ref[...])


def _tc_select(fringe, minf, status):
    spec = pl.BlockSpec((TC_BLOCK,), lambda i: (i,))
    return pl.pallas_call(
        _select_body,
        grid=(TC_GRID,),
        in_specs=[spec, spec, spec],
        out_specs=spec,
        out_shape=jax.ShapeDtypeStruct((N_NODES,), jnp.float32),
    )(fringe, minf, status)


def _flux_body(step_ref, part_ref, flux_ref, out_ref):
    total = jnp.sum(part_ref[...])
    step = step_ref[0, 0]
    rows = lax.broadcasted_iota(jnp.int32, (8, 125), 0)
    cols = lax.broadcasted_iota(jnp.int32, (8, 125), 1)
    flat_idx = rows * 125 + cols
    out_ref[...] = jnp.where(flat_idx == step, total, flux_ref[...])


def _flux_update(step2d, partials, flux2d):
    return pl.pallas_call(
        _flux_body,
        out_shape=jax.ShapeDtypeStruct((8, 125), jnp.float32),
        in_specs=[
            pl.BlockSpec(memory_space=pltpu.SMEM),
            pl.BlockSpec(memory_space=pltpu.VMEM),
            pl.BlockSpec(memory_space=pltpu.VMEM),
        ],
        out_specs=pl.BlockSpec(memory_space=pltpu.VMEM),
    )(step2d, partials, flux2d)


def kernel(fringe_thickness, min_fringe_thickness, fluxes, node_is_terminus,
           status_at_node, current_step):
    partials = _sc_reduce(fringe_thickness, node_is_terminus)
    cleared = _tc_select(fringe_thickness, min_fringe_thickness, status_at_node)
    step2d = jnp.asarray(current_step, jnp.int32).reshape(1, 1)
    flux2d = fluxes.reshape(8, 125)
    out2d = _flux_update(step2d, partials, flux2d)
    return cleared, out2d.reshape(fluxes.shape)


# speedup vs baseline: 1.9169x; 1.9169x over previous
import functools

import jax
import jax.numpy as jnp
from jax import lax
from jax.experimental import pallas as pl
from jax.experimental.pallas import tpu as pltpu
from jax.experimental.pallas import tpu_sc as plsc

N_NODES = 1_000_000
CHUNK = 8_000
N_CHUNKS = N_NODES // CHUNK
N_WORKERS = 32
MAX_ITERS = -(-N_CHUNKS // N_WORKERS)
LANES = 16
GROUPS = 4


def _sc_reduce_body(fringe_hbm, term_hbm, part_hbm,
                    f0, f1, t0, t1, acc_v, isem0, isem1):
    wid = lax.axis_index("s") * 2 + lax.axis_index("c")
    f_v, t_v = (f0, f1), (t0, t1)
    isems = (isem0, isem1)
    acc_v[...] = jnp.zeros((LANES,), jnp.float32)

    def in_copies(slot, chunk):
        off = chunk * CHUNK
        sl = pl.ds(off, CHUNK)
        return (
            pltpu.make_async_copy(fringe_hbm.at[sl], f_v[slot], isems[slot]),
            pltpu.make_async_copy(term_hbm.at[sl], t_v[slot], isems[slot]),
        )

    for it in range(MAX_ITERS):
        chunk = wid + it * N_WORKERS
        slot = it % 2

        if it == 0:
            @pl.when(chunk < N_CHUNKS)
            def _prime():
                for c in in_copies(0, chunk):
                    c.start()

        if it + 1 < MAX_ITERS:
            nxt = chunk + N_WORKERS

            @pl.when(nxt < N_CHUNKS)
            def _prefetch():
                for c in in_copies(1 - slot, nxt):
                    c.start()

        @pl.when(chunk < N_CHUNKS)
        def _process():
            for c in in_copies(slot, chunk):
                c.wait()

            zero = jnp.zeros((LANES,), jnp.float32)

            @plsc.parallel_loop(0, CHUNK, step=GROUPS * LANES, unroll=2,
                                carry=(zero, zero, zero, zero))
            def body(j, accs):
                new = []
                for g in range(GROUPS):
                    sl = pl.ds(j + g * LANES, LANES)
                    new.append(accs[g]
                               + f_v[slot][sl] * t_v[slot][sl].astype(jnp.float32))
                return tuple(new)

            a0, a1, a2, a3 = body
            acc_v[...] = acc_v[...] + ((a0 + a1) + (a2 + a3))

    pltpu.sync_copy(acc_v, part_hbm.at[wid])


@functools.partial(
    pl.kernel,
    out_type=jax.ShapeDtypeStruct((N_WORKERS, LANES), jnp.float32),
    mesh=plsc.VectorSubcoreMesh(core_axis_name="c", subcore_axis_name="s"),
    scratch_types=[
        pltpu.VMEM((CHUNK,), jnp.float32),
        pltpu.VMEM((CHUNK,), jnp.float32),
        pltpu.VMEM((CHUNK,), jnp.int32),
        pltpu.VMEM((CHUNK,), jnp.int32),
        pltpu.VMEM((LANES,), jnp.float32),
        pltpu.SemaphoreType.DMA,
        pltpu.SemaphoreType.DMA,
    ],
)
def _sc_reduce(*args):
    _sc_reduce_body(*args)


TC_BLOCK = 131_072
TC_GRID = -(-N_NODES // TC_BLOCK)


def _select_body(f_ref, m_ref, s_ref, o_ref):
    o_ref[...] = jnp.where(s_ref[...] == 0, f_ref[...], m_ref[...])


def _tc_select(fringe, minf, status):
    spec = pl.BlockSpec((TC_BLOCK,), lambda i: (i,))
    return pl.pallas_call(
        _select_body,
        grid=(TC_GRID,),
        in_specs=[spec, spec, spec],
        out_specs=spec,
        out_shape=jax.ShapeDtypeStruct((N_NODES,), jnp.float32),
    )(fringe, minf, status)


def _flux_body(step_ref, part_ref, flux_ref, out_ref):
    total = jnp.sum(part_ref[...])
    step = step_ref[0, 0]
    cols = lax.broadcasted_iota(jnp.int32, (1, 1000), 1)
    out_ref[...] = jnp.where(cols == step, total, flux_ref[...])


def _flux_update(step2d, partials, flux2d):
    return pl.pallas_call(
        _flux_body,
        out_shape=jax.ShapeDtypeStruct((1, 1000), jnp.float32),
        in_specs=[
            pl.BlockSpec(memory_space=pltpu.SMEM),
            pl.BlockSpec(memory_space=pltpu.VMEM),
            pl.BlockSpec(memory_space=pltpu.VMEM),
        ],
        out_specs=pl.BlockSpec(memory_space=pltpu.VMEM),
    )(step2d, partials, flux2d)


def kernel(fringe_thickness, min_fringe_thickness, fluxes, node_is_terminus,
           status_at_node, current_step):
    partials = _sc_reduce(fringe_thickness, node_is_terminus)
    cleared = _tc_select(fringe_thickness, min_fringe_thickness, status_at_node)
    step2d = jnp.asarray(current_step, jnp.int32).reshape(1, 1)
    flux2d = fluxes.reshape(1, 1000)
    out2d = _flux_update(step2d, partials, flux2d)
    return cleared, out2d.reshape(fluxes.shape)
